# baseline (speedup 1.0000x reference)
import functools

import jax
import jax.numpy as jnp
from jax import lax
from jax.experimental import pallas as pl
from jax.experimental.pallas import tpu as pltpu
from jax.experimental.pallas import tpu_sc as plsc

NUM_EXPERTS = 8
TOP_K = 2
HIDDEN = 768
INTER = 3072
TOKENS = 2048

BM = 256
PADDED = TOP_K * TOKENS + NUM_EXPERTS * BM
NBLK = PADDED // BM
KSPLIT = 2
IB = INTER // KSPLIT


def _mlp_block_kernel(be_ref, xs_ref, w1_ref, w3_ref, w2_ref, ws_ref, o_ref):
    k = pl.program_id(1)
    x = xs_ref[...]
    w1b = w1_ref[0]
    w3b = w3_ref[0]
    w2b = w2_ref[0]
    gate = jax.lax.dot_general(x, w1b, (((1,), (1,)), ((), ())),
                               preferred_element_type=jnp.float32)
    up = jax.lax.dot_general(x, w3b, (((1,), (1,)), ((), ())),
                             preferred_element_type=jnp.float32)
    h = gate * jax.nn.sigmoid(gate) * up
    o = jax.lax.dot_general(h, w2b, (((1,), (1,)), ((), ())),
                            preferred_element_type=jnp.float32)
    o = o * ws_ref[0, 0][:, None]

    @pl.when(k == 0)
    def _():
        o_ref[...] = o

    @pl.when(k != 0)
    def _():
        o_ref[...] += o


def _grouped_mlp(xs, w1, w3, w2, ws3d, block_expert):
    def kk(i, k):
        return jax.lax.bitwise_xor(k, i % 2)

    grid_spec = pltpu.PrefetchScalarGridSpec(
        num_scalar_prefetch=1,
        grid=(NBLK, KSPLIT),
        in_specs=[
            pl.BlockSpec((BM, HIDDEN), lambda i, k, be: (i, 0)),
            pl.BlockSpec((1, IB, HIDDEN), lambda i, k, be: (be[i], kk(i, k), 0)),
            pl.BlockSpec((1, IB, HIDDEN), lambda i, k, be: (be[i], kk(i, k), 0)),
            pl.BlockSpec((1, HIDDEN, IB), lambda i, k, be: (be[i], 0, kk(i, k))),
            pl.BlockSpec((1, 1, BM), lambda i, k, be: (i, 0, 0)),
        ],
        out_specs=pl.BlockSpec((BM, HIDDEN), lambda i, k, be: (i, 0)),
    )
    return pl.pallas_call(
        _mlp_block_kernel,
        grid_spec=grid_spec,
        out_shape=jax.ShapeDtypeStruct((PADDED, HIDDEN), jnp.float32),
        compiler_params=pltpu.CompilerParams(
            dimension_semantics=("arbitrary", "arbitrary")),
    )(block_expert, xs, w1, w3, w2, ws3d)


_NC, _NS = 2, 16
_NW = _NC * _NS
_SLOTS_PER_W = PADDED // _NW
_GCHUNK = 64
_TOK_PER_W = TOKENS // _NW
_CCHUNK = 16
_POSPAD = 2 * TOKENS + 128


def _sc_mesh():
    return plsc.VectorSubcoreMesh(core_axis_name="c", subcore_axis_name="s")


def _gather_rows(x, idx, n_rows):
    per_w = n_rows // _NW
    nch = per_w // _GCHUNK

    @functools.partial(
        pl.kernel,
        out_type=jax.ShapeDtypeStruct((n_rows, HIDDEN), jnp.float32),
        mesh=_sc_mesh(),
        scratch_types=[
            pltpu.VMEM((2, _GCHUNK), jnp.int32),
            pltpu.VMEM((_GCHUNK, HIDDEN), jnp.float32),
            pltpu.VMEM((_GCHUNK, HIDDEN), jnp.float32),
            pltpu.SemaphoreType.DMA,
            pltpu.SemaphoreType.DMA,
            pltpu.SemaphoreType.DMA,
            pltpu.SemaphoreType.DMA,
        ],
    )
    def k(x_hbm, tok_hbm, xs_hbm, idx_v, rows0, rows1, g0, g1, w0, w1):
        wid = lax.axis_index("s") * _NC + lax.axis_index("c")
        base = wid * per_w
        rows = (rows0, rows1)
        gsem = (g0, g1)
        wsem = (w0, w1)
        gathers = [None] * nch
        writes = [None] * nch
        for c in range(nch):
            off = base + c * _GCHUNK
            pltpu.sync_copy(tok_hbm.at[pl.ds(off, _GCHUNK)], idx_v.at[c % 2])
            if c >= 2:
                writes[c - 2].wait()
            gathers[c] = pltpu.async_copy(
                x_hbm.at[idx_v.at[c % 2]], rows[c % 2], gsem[c % 2])
            gathers[c].wait()
            writes[c] = pltpu.async_copy(
                rows[c % 2], xs_hbm.at[pl.ds(off, _GCHUNK)], wsem[c % 2])
        for c in range(max(0, nch - 2), nch):
            writes[c].wait()

    return k(x, idx)


_POSDUMP = 2 * TOKENS
_TPT = TOKENS // _NS
_ZPT = PADDED // _NS


def _route_sort(logits_t):
    @functools.partial(
        pl.kernel,
        out_type=(
            jax.ShapeDtypeStruct((PADDED,), jnp.int32),
            jax.ShapeDtypeStruct((PADDED,), jnp.float32),
            jax.ShapeDtypeStruct((_POSPAD,), jnp.int32),
            jax.ShapeDtypeStruct((32,), jnp.int32),
        ),
        mesh=_sc_mesh(),
        compiler_params=pltpu.CompilerParams(needs_layout_passes=False),
        scratch_types=[
            pltpu.VMEM((8, _TPT), jnp.float32),
            pltpu.VMEM((_TPT,), jnp.int32),
            pltpu.VMEM((_TPT,), jnp.int32),
            pltpu.VMEM((_TPT,), jnp.float32),
            pltpu.VMEM((_TPT,), jnp.float32),
            pltpu.VMEM((_ZPT,), jnp.int32),
            pltpu.VMEM((_ZPT,), jnp.float32),
            pltpu.VMEM((2 * TOKENS,), jnp.int32),
            pltpu.VMEM((2 * TOKENS,), jnp.float32),
            pltpu.VMEM((16,), jnp.int32),
            pltpu.VMEM((2 * TOKENS // 2,), jnp.int32),
            pltpu.VMEM((2 * TOKENS // 2,), jnp.float32),
            pltpu.VMEM((16, 128), jnp.int32),
            pltpu.VMEM((16, 128), jnp.int32),
            pltpu.VMEM((32,), jnp.int32),
            pltpu.VMEM((16,), jnp.int32),
            pltpu.VMEM((16,), jnp.int32),
            pltpu.VMEM_SHARED((2 * TOKENS,), jnp.int32),
            pltpu.VMEM_SHARED((2 * TOKENS,), jnp.float32),
            pltpu.SemaphoreType.DMA,
        ],
    )
    def k(lt_hbm, tok_hbm, ws_hbm, pos_hbm, be_hbm,
          lv, e1b, e2b, p1b, p2b, zt, zw, ef_v, pf_v, acc,
          tokbuf, wsbuf, plidx, valb, bebuf, psum, off_ref,
          ef_sh, pf_sh, sem):
        cid = lax.axis_index("c")
        sid = lax.axis_index("s")
        iota = lax.iota(jnp.int32, 16)
        ones = jnp.ones((16,), jnp.int32)

        @pl.when(cid == 0)
        def _core0():
            for cc in range(_ZPT // 16):
                base = sid * _ZPT + cc * 16
                zt[pl.ds(cc * 16, 16)] = ((base + iota) * 997) & (TOKENS - 1)
                zw[pl.ds(cc * 16, 16)] = jnp.zeros((16,), jnp.float32)
            pltpu.sync_copy(zt, tok_hbm.at[pl.ds(pl.multiple_of(sid * _ZPT, 8), _ZPT)])
            pltpu.sync_copy(zw, ws_hbm.at[pl.ds(pl.multiple_of(sid * _ZPT, 8), _ZPT)])

            t0 = sid * _TPT
            pltpu.sync_copy(lt_hbm.at[:, pl.ds(t0, _TPT)], lv)
            for c8 in range(_TPT // 16):
                sl = pl.ds(c8 * 16, 16)
                l = [lv[e, sl] for e in range(NUM_EXPERTS)]
                m1 = l[0]
                for e in range(1, NUM_EXPERTS):
                    m1 = jnp.maximum(m1, l[e])
                e1 = jnp.full((16,), NUM_EXPERTS, jnp.int32)
                for e in range(NUM_EXPERTS):
                    e1 = jnp.minimum(
                        e1, jnp.where(l[e] >= m1, e, NUM_EXPERTS))
                l2 = [jnp.where(e1 == e, -1e30, l[e])
                      for e in range(NUM_EXPERTS)]
                m2 = l2[0]
                for e in range(1, NUM_EXPERTS):
                    m2 = jnp.maximum(m2, l2[e])
                e2 = jnp.full((16,), NUM_EXPERTS, jnp.int32)
                for e in range(NUM_EXPERTS):
                    e2 = jnp.minimum(
                        e2, jnp.where(l2[e] >= m2, e, NUM_EXPERTS))
                ex = jnp.exp(m2 - m1)
                p1 = 1.0 / (1.0 + ex)
                e1b[sl] = e1
                e2b[sl] = e2
                p1b[sl] = p1
                p2b[sl] = 1.0 - p1
            pltpu.sync_copy(e1b, ef_sh.at[pl.ds(pl.multiple_of(t0, 8), _TPT)])
            pltpu.sync_copy(e2b, ef_sh.at[pl.ds(pl.multiple_of(TOKENS + t0, 8), _TPT)])
            pltpu.sync_copy(p1b, pf_sh.at[pl.ds(pl.multiple_of(t0, 8), _TPT)])
            pltpu.sync_copy(p2b, pf_sh.at[pl.ds(pl.multiple_of(TOKENS + t0, 8), _TPT)])
            plsc.subcore_barrier()

            @pl.when(sid <= NUM_EXPERTS)
            def _p2a():
                pltpu.sync_copy(ef_sh, ef_v)
                pltpu.sync_copy(pf_sh, pf_v)
                acc[...] = jnp.zeros((16,), jnp.int32)

                def hbody(c, _):
                    ve = ef_v[pl.ds(pl.multiple_of(c * 16, 16), 16)]
                    plsc.addupdate_scatter(acc, [ve], ones)
                    return 0

                lax.fori_loop(0, 2 * TOKENS // 16, hbody, 0)
                cv = acc[...]
                run = jnp.int32(0)
                off_vec = jnp.zeros((16,), jnp.int32)
                for e in range(NUM_EXPERTS):
                    off_vec = off_vec + jnp.where(iota == e, run, 0)
                    run = run + (((cv[e] + (BM - 1)) >> 8) << 8)
                off_ref[...] = off_vec

                @pl.when(sid < NUM_EXPERTS)
                def _p2b():
                    sidv = jnp.zeros((16,), jnp.int32) + sid
                    my_base = plsc.load_gather(off_ref, [sidv])[0]
                    for r in range(16):
                        for cc in range(8):
                            plidx[r, pl.ds(cc * 16, 16)] = jnp.full(
                                (16,), _POSDUMP, jnp.int32)
                            valb[r, pl.ds(cc * 16, 16)] = (
                                my_base + r * 128 + cc * 16 + iota)
                    for cc in range(128):
                        sl = pl.ds(cc * 16, 16)
                        tokbuf[sl] = ((my_base + cc * 16 + iota) * 997) \
                            & (TOKENS - 1)
                        wsbuf[sl] = jnp.zeros((16,), jnp.float32)

                    def sbody(c, cnt_vec):
                        sl = pl.ds(pl.multiple_of(c * 16, 16), 16)
                        ve = ef_v[sl]
                        wv = pf_v[sl]
                        tc = c * 16 + iota
                        mask = ve == sid
                        mi = jnp.where(mask, 1, 0).astype(jnp.int32)
                        ps = mi
                        for d in (1, 2, 4, 8):
                            psum[...] = ps
                            g = plsc.load_gather(
                                psum, [jnp.maximum(iota - d, 0)])
                            ps = ps + jnp.where(iota >= d, g, 0)
                        rk = ps - 1
                        slotv = cnt_vec + rk
                        tvec = tc & (TOKENS - 1)
                        plsc.store_scatter(tokbuf, [slotv], tvec, mask=mask)
                        plsc.store_scatter(wsbuf, [slotv], wv, mask=mask)
                        plsc.store_scatter(
                            plidx, [slotv >> 7, slotv & 127], tc, mask=mask)
                        psum[...] = ps
                        lane15 = jnp.full((16,), 15, jnp.int32)
                        tot = plsc.load_gather(psum, [lane15])
                        return cnt_vec + tot

                    cnt_vec = lax.fori_loop(
                        0, 2 * TOKENS // 16, sbody,
                        jnp.zeros((16,), jnp.int32))
                    cnt = cnt_vec[0]

                    def wbody(j, _):
                        src = pl.ds(pl.multiple_of(j * BM, BM), BM)
                        dst = pl.ds(
                            pl.multiple_of(my_base + j * BM, BM), BM)
                        pltpu.sync_copy(tokbuf.at[src], tok_hbm.at[dst])
                        pltpu.sync_copy(wsbuf.at[src], ws_hbm.at[dst])
                        return 0

                    lax.fori_loop(0, (cnt + BM - 1) >> 8, wbody, 0)
                    copies = []
                    for r in range(16):
                        copies.append(pltpu.async_copy(
                            valb.at[r], pos_hbm.at[plidx.at[r]], sem))
                    for cp in copies:
                        cp.wait()

                @pl.when(sid == NUM_EXPERTS)
                def _be():
                    for c2 in range(2):
                        blk = (iota + c2 * 16) * BM
                        accv = jnp.zeros((16,), jnp.int32)
                        for e in range(NUM_EXPERTS):
                            accv = accv + jnp.where(
                                blk >= off_vec[e], 1, 0)
                        bebuf[pl.ds(c2 * 16, 16)] = jnp.minimum(
                            jnp.maximum(accv - 1, 0), NUM_EXPERTS - 1)
                    pltpu.sync_copy(bebuf, be_hbm)

    return k(logits_t)


def _pair_add_kernel(a_ref, b_ref, o_ref):
    o_ref[...] = a_ref[...] + b_ref[...]


def _combine_rows(o_sorted, pos):
    pairs = _gather_rows(o_sorted, pos, 2 * TOKENS)
    badd = 512
    nb = TOKENS // badd
    return pl.pallas_call(
        _pair_add_kernel,
        grid=(nb,),
        in_specs=[
            pl.BlockSpec((badd, HIDDEN), lambda i: (i, 0)),
            pl.BlockSpec((badd, HIDDEN), lambda i: (i + nb, 0)),
        ],
        out_specs=pl.BlockSpec((badd, HIDDEN), lambda i: (i, 0)),
        out_shape=jax.ShapeDtypeStruct((TOKENS, HIDDEN), jnp.float32),
    )(pairs, pairs)


def kernel(hidden_states, router_logits, w1, w3, w2):
    x = hidden_states.reshape(-1, HIDDEN)

    tok_sorted, ws_buf, pos, block_expert = _route_sort(router_logits.T)

    xs = _gather_rows(x, tok_sorted, PADDED)

    ws3d = ws_buf.reshape(NBLK, 1, BM)
    o_sorted = _grouped_mlp(xs, w1, w3, w2, ws3d, block_expert)

    out = _combine_rows(o_sorted, pos)
    return out


# speedup vs baseline: 6.7205x; 6.7205x over previous
import functools

import jax
import jax.numpy as jnp
from jax import lax
from jax.experimental import pallas as pl
from jax.experimental.pallas import tpu as pltpu
from jax.experimental.pallas import tpu_sc as plsc

NUM_EXPERTS = 8
TOP_K = 2
HIDDEN = 768
INTER = 3072
TOKENS = 2048

BM = 256
PADDED = TOP_K * TOKENS + NUM_EXPERTS * BM
NBLK = PADDED // BM
KSPLIT = 2
IB = INTER // KSPLIT


def _mlp_block_kernel(be_ref, xs_ref, w1_ref, w3_ref, w2_ref, ws_ref, o_ref):
    k = pl.program_id(1)
    x = xs_ref[...]
    w1b = w1_ref[0]
    w3b = w3_ref[0]
    w2b = w2_ref[0]
    gate = jax.lax.dot_general(x, w1b, (((1,), (1,)), ((), ())),
                               preferred_element_type=jnp.float32)
    up = jax.lax.dot_general(x, w3b, (((1,), (1,)), ((), ())),
                             preferred_element_type=jnp.float32)
    h = gate * jax.nn.sigmoid(gate) * up
    o = jax.lax.dot_general(h, w2b, (((1,), (1,)), ((), ())),
                            preferred_element_type=jnp.float32)
    o = o * ws_ref[0, 0][:, None]

    @pl.when(k == 0)
    def _():
        o_ref[...] = o

    @pl.when(k != 0)
    def _():
        o_ref[...] += o


def _grouped_mlp(xs, w1, w3, w2, ws3d, block_expert):
    def kk(i, k):
        return jax.lax.bitwise_xor(k, i % 2)

    grid_spec = pltpu.PrefetchScalarGridSpec(
        num_scalar_prefetch=1,
        grid=(NBLK, KSPLIT),
        in_specs=[
            pl.BlockSpec((BM, HIDDEN), lambda i, k, be: (i, 0)),
            pl.BlockSpec((1, IB, HIDDEN), lambda i, k, be: (be[i], kk(i, k), 0)),
            pl.BlockSpec((1, IB, HIDDEN), lambda i, k, be: (be[i], kk(i, k), 0)),
            pl.BlockSpec((1, HIDDEN, IB), lambda i, k, be: (be[i], 0, kk(i, k))),
            pl.BlockSpec((1, 1, BM), lambda i, k, be: (i, 0, 0)),
        ],
        out_specs=pl.BlockSpec((BM, HIDDEN), lambda i, k, be: (i, 0)),
    )
    return pl.pallas_call(
        _mlp_block_kernel,
        grid_spec=grid_spec,
        out_shape=jax.ShapeDtypeStruct((PADDED, HIDDEN), jnp.float32),
        compiler_params=pltpu.CompilerParams(
            dimension_semantics=("arbitrary", "arbitrary")),
    )(block_expert, xs, w1, w3, w2, ws3d)


_NC, _NS = 2, 16
_NW = _NC * _NS
_SLOTS_PER_W = PADDED // _NW
_GCHUNK = 64
_TOK_PER_W = TOKENS // _NW
_CCHUNK = 16
_POSPAD = 2 * TOKENS + NUM_EXPERTS * 256


def _sc_mesh():
    return plsc.VectorSubcoreMesh(core_axis_name="c", subcore_axis_name="s")


def _gather_rows(x, idx, n_rows):
    per_w = n_rows // _NW
    nch = per_w // _GCHUNK

    @functools.partial(
        pl.kernel,
        out_type=jax.ShapeDtypeStruct((n_rows, HIDDEN), jnp.float32),
        mesh=_sc_mesh(),
        scratch_types=[
            pltpu.VMEM((2, _GCHUNK), jnp.int32),
            pltpu.VMEM((_GCHUNK, HIDDEN), jnp.float32),
            pltpu.VMEM((_GCHUNK, HIDDEN), jnp.float32),
            pltpu.SemaphoreType.DMA,
            pltpu.SemaphoreType.DMA,
            pltpu.SemaphoreType.DMA,
            pltpu.SemaphoreType.DMA,
        ],
    )
    def k(x_hbm, tok_hbm, xs_hbm, idx_v, rows0, rows1, g0, g1, w0, w1):
        wid = lax.axis_index("s") * _NC + lax.axis_index("c")
        base = wid * per_w
        rows = (rows0, rows1)
        gsem = (g0, g1)
        wsem = (w0, w1)
        gathers = [None] * nch
        writes = [None] * nch
        for c in range(nch):
            off = base + c * _GCHUNK
            pltpu.sync_copy(tok_hbm.at[pl.ds(off, _GCHUNK)], idx_v.at[c % 2])
            if c >= 2:
                writes[c - 2].wait()
            gathers[c] = pltpu.async_copy(
                x_hbm.at[idx_v.at[c % 2]], rows[c % 2], gsem[c % 2])
            gathers[c].wait()
            writes[c] = pltpu.async_copy(
                rows[c % 2], xs_hbm.at[pl.ds(off, _GCHUNK)], wsem[c % 2])
        for c in range(max(0, nch - 2), nch):
            writes[c].wait()

    return k(x, idx)


_POSDUMP = 2 * TOKENS
_TPT = TOKENS // _NS
_ZPT = PADDED // _NS


def _route_sort(logits_t):
    @functools.partial(
        pl.kernel,
        out_type=(
            jax.ShapeDtypeStruct((PADDED,), jnp.int32),
            jax.ShapeDtypeStruct((PADDED,), jnp.float32),
            jax.ShapeDtypeStruct((_POSPAD,), jnp.int32),
            jax.ShapeDtypeStruct((32,), jnp.int32),
        ),
        mesh=_sc_mesh(),
        compiler_params=pltpu.CompilerParams(needs_layout_passes=False),
        scratch_types=[
            pltpu.VMEM((8, _TPT), jnp.float32),
            pltpu.VMEM((_TPT,), jnp.int32),
            pltpu.VMEM((_TPT,), jnp.int32),
            pltpu.VMEM((_TPT,), jnp.float32),
            pltpu.VMEM((_TPT,), jnp.float32),
            pltpu.VMEM((_ZPT,), jnp.int32),
            pltpu.VMEM((_ZPT,), jnp.float32),
            pltpu.VMEM((2 * TOKENS,), jnp.int32),
            pltpu.VMEM((2 * TOKENS,), jnp.float32),
            pltpu.VMEM((16,), jnp.int32),
            pltpu.VMEM((2 * TOKENS // 2,), jnp.int32),
            pltpu.VMEM((2 * TOKENS // 2,), jnp.float32),
            pltpu.VMEM((16, 128), jnp.int32),
            pltpu.VMEM((16, 128), jnp.int32),
            pltpu.VMEM((32,), jnp.int32),
            pltpu.VMEM((16,), jnp.int32),
            pltpu.VMEM((16,), jnp.int32),
            pltpu.VMEM_SHARED((2 * TOKENS,), jnp.int32),
            pltpu.VMEM_SHARED((2 * TOKENS,), jnp.float32),
            pltpu.SemaphoreType.DMA,
        ],
    )
    def k(lt_hbm, tok_hbm, ws_hbm, pos_hbm, be_hbm,
          lv, e1b, e2b, p1b, p2b, zt, zw, ef_v, pf_v, acc,
          tokbuf, wsbuf, plidx, valb, bebuf, psum, off_ref,
          ef_sh, pf_sh, sem):
        cid = lax.axis_index("c")
        sid = lax.axis_index("s")
        iota = lax.iota(jnp.int32, 16)
        ones = jnp.ones((16,), jnp.int32)

        @pl.when(cid == 0)
        def _core0():
            for cc in range(_ZPT // 16):
                base = sid * _ZPT + cc * 16
                zt[pl.ds(cc * 16, 16)] = ((base + iota) * 997) & (TOKENS - 1)
                zw[pl.ds(cc * 16, 16)] = jnp.zeros((16,), jnp.float32)
            pltpu.sync_copy(zt, tok_hbm.at[pl.ds(pl.multiple_of(sid * _ZPT, 8), _ZPT)])
            pltpu.sync_copy(zw, ws_hbm.at[pl.ds(pl.multiple_of(sid * _ZPT, 8), _ZPT)])

            t0 = sid * _TPT
            pltpu.sync_copy(lt_hbm.at[:, pl.ds(t0, _TPT)], lv)
            for c8 in range(_TPT // 16):
                sl = pl.ds(c8 * 16, 16)
                l = [lv[e, sl] for e in range(NUM_EXPERTS)]
                m1 = l[0]
                for e in range(1, NUM_EXPERTS):
                    m1 = jnp.maximum(m1, l[e])
                e1 = jnp.full((16,), NUM_EXPERTS, jnp.int32)
                for e in range(NUM_EXPERTS):
                    e1 = jnp.minimum(
                        e1, jnp.where(l[e] >= m1, e, NUM_EXPERTS))
                l2 = [jnp.where(e1 == e, -1e30, l[e])
                      for e in range(NUM_EXPERTS)]
                m2 = l2[0]
                for e in range(1, NUM_EXPERTS):
                    m2 = jnp.maximum(m2, l2[e])
                e2 = jnp.full((16,), NUM_EXPERTS, jnp.int32)
                for e in range(NUM_EXPERTS):
                    e2 = jnp.minimum(
                        e2, jnp.where(l2[e] >= m2, e, NUM_EXPERTS))
                ex = jnp.exp(m2 - m1)
                p1 = 1.0 / (1.0 + ex)
                e1b[sl] = e1
                e2b[sl] = e2
                p1b[sl] = p1
                p2b[sl] = 1.0 - p1
            pltpu.sync_copy(e1b, ef_sh.at[pl.ds(pl.multiple_of(t0, 8), _TPT)])
            pltpu.sync_copy(e2b, ef_sh.at[pl.ds(pl.multiple_of(TOKENS + t0, 8), _TPT)])
            pltpu.sync_copy(p1b, pf_sh.at[pl.ds(pl.multiple_of(t0, 8), _TPT)])
            pltpu.sync_copy(p2b, pf_sh.at[pl.ds(pl.multiple_of(TOKENS + t0, 8), _TPT)])
            plsc.subcore_barrier()

            @pl.when(sid <= NUM_EXPERTS)
            def _p2a():
                pltpu.sync_copy(ef_sh, ef_v)
                pltpu.sync_copy(pf_sh, pf_v)
                acc[...] = jnp.zeros((16,), jnp.int32)

                def hbody(c, _):
                    ve = ef_v[pl.ds(pl.multiple_of(c * 16, 16), 16)]
                    plsc.addupdate_scatter(acc, [ve], ones)
                    return 0

                lax.fori_loop(0, 2 * TOKENS // 16, hbody, 0)
                cv = acc[...]
                run = jnp.int32(0)
                off_vec = jnp.zeros((16,), jnp.int32)
                for e in range(NUM_EXPERTS):
                    off_vec = off_vec + jnp.where(iota == e, run, 0)
                    run = run + (((cv[e] + (BM - 1)) >> 8) << 8)
                off_ref[...] = off_vec

                @pl.when(sid < NUM_EXPERTS)
                def _p2b():
                    sidv = jnp.zeros((16,), jnp.int32) + sid
                    my_base = plsc.load_gather(off_ref, [sidv])[0]
                    dumpb = _POSDUMP + sid * 256
                    for r in range(16):
                        for cc in range(8):
                            plidx[r, pl.ds(cc * 16, 16)] = (
                                dumpb + (r % 2) * 128 + cc * 16 + iota)
                            valb[r, pl.ds(cc * 16, 16)] = (
                                my_base + r * 128 + cc * 16 + iota)
                    for cc in range(128):
                        sl = pl.ds(cc * 16, 16)
                        tokbuf[sl] = ((my_base + cc * 16 + iota) * 997) \
                            & (TOKENS - 1)
                        wsbuf[sl] = jnp.zeros((16,), jnp.float32)

                    def sbody(c, cnt_vec):
                        sl = pl.ds(pl.multiple_of(c * 16, 16), 16)
                        ve = ef_v[sl]
                        wv = pf_v[sl]
                        tc = c * 16 + iota
                        mask = ve == sid
                        mi = jnp.where(mask, 1, 0).astype(jnp.int32)
                        ps = mi
                        for d in (1, 2, 4, 8):
                            psum[...] = ps
                            g = plsc.load_gather(
                                psum, [jnp.maximum(iota - d, 0)])
                            ps = ps + jnp.where(iota >= d, g, 0)
                        rk = ps - 1
                        slotv = cnt_vec + rk
                        tvec = tc & (TOKENS - 1)
                        plsc.store_scatter(tokbuf, [slotv], tvec, mask=mask)
                        plsc.store_scatter(wsbuf, [slotv], wv, mask=mask)
                        plsc.store_scatter(
                            plidx, [slotv >> 7, slotv & 127], tc, mask=mask)
                        psum[...] = ps
                        lane15 = jnp.full((16,), 15, jnp.int32)
                        tot = plsc.load_gather(psum, [lane15])
                        return cnt_vec + tot

                    cnt_vec = lax.fori_loop(
                        0, 2 * TOKENS // 16, sbody,
                        jnp.zeros((16,), jnp.int32))
                    cnt = cnt_vec[0]

                    def wbody(j, _):
                        src = pl.ds(pl.multiple_of(j * BM, BM), BM)
                        dst = pl.ds(
                            pl.multiple_of(my_base + j * BM, BM), BM)
                        pltpu.sync_copy(tokbuf.at[src], tok_hbm.at[dst])
                        pltpu.sync_copy(wsbuf.at[src], ws_hbm.at[dst])
                        return 0

                    lax.fori_loop(0, (cnt + BM - 1) >> 8, wbody, 0)

                    def pbody(r, _):
                        pltpu.sync_copy(valb.at[r], pos_hbm.at[plidx.at[r]])
                        return 0

                    lax.fori_loop(0, (cnt + 127) >> 7, pbody, 0)

                @pl.when(sid == NUM_EXPERTS)
                def _be():
                    for c2 in range(2):
                        blk = (iota + c2 * 16) * BM
                        accv = jnp.zeros((16,), jnp.int32)
                        for e in range(NUM_EXPERTS):
                            accv = accv + jnp.where(
                                blk >= off_vec[e], 1, 0)
                        bebuf[pl.ds(c2 * 16, 16)] = jnp.minimum(
                            jnp.maximum(accv - 1, 0), NUM_EXPERTS - 1)
                    pltpu.sync_copy(bebuf, be_hbm)

    return k(logits_t)


def _pair_add_kernel(a_ref, b_ref, o_ref):
    o_ref[...] = a_ref[...] + b_ref[...]


def _combine_rows(o_sorted, pos):
    pairs = _gather_rows(o_sorted, pos, 2 * TOKENS)
    badd = 512
    nb = TOKENS // badd
    return pl.pallas_call(
        _pair_add_kernel,
        grid=(nb,),
        in_specs=[
            pl.BlockSpec((badd, HIDDEN), lambda i: (i, 0)),
            pl.BlockSpec((badd, HIDDEN), lambda i: (i + nb, 0)),
        ],
        out_specs=pl.BlockSpec((badd, HIDDEN), lambda i: (i, 0)),
        out_shape=jax.ShapeDtypeStruct((TOKENS, HIDDEN), jnp.float32),
    )(pairs, pairs)


def kernel(hidden_states, router_logits, w1, w3, w2):
    x = hidden_states.reshape(-1, HIDDEN)

    tok_sorted, ws_buf, pos, block_expert = _route_sort(router_logits.T)

    xs = _gather_rows(x, tok_sorted, PADDED)

    ws3d = ws_buf.reshape(NBLK, 1, BM)
    o_sorted = _grouped_mlp(xs, w1, w3, w2, ws3d, block_expert)

    out = _combine_rows(o_sorted, pos)
    return out


# speedup vs baseline: 6.8602x; 1.0208x over previous
import functools

import jax
import jax.numpy as jnp
from jax import lax
from jax.experimental import pallas as pl
from jax.experimental.pallas import tpu as pltpu
from jax.experimental.pallas import tpu_sc as plsc

NUM_EXPERTS = 8
TOP_K = 2
HIDDEN = 768
INTER = 3072
TOKENS = 2048

BM = 256
PADDED = TOP_K * TOKENS + NUM_EXPERTS * BM
NBLK = PADDED // BM
KSPLIT = 2
IB = INTER // KSPLIT


def _mlp_block_kernel(be_ref, xs_ref, w1_ref, w3_ref, w2_ref, ws_ref, o_ref):
    k = pl.program_id(1)
    i = pl.program_id(0)
    live = be_ref[i] < NUM_EXPERTS

    @pl.when(live)
    def _body():
        _mlp_live(k, xs_ref, w1_ref, w3_ref, w2_ref, ws_ref, o_ref)


def _mlp_live(k, xs_ref, w1_ref, w3_ref, w2_ref, ws_ref, o_ref):
    x = xs_ref[...]
    w1b = w1_ref[0]
    w3b = w3_ref[0]
    w2b = w2_ref[0]
    gate = jax.lax.dot_general(x, w1b, (((1,), (1,)), ((), ())),
                               preferred_element_type=jnp.float32)
    up = jax.lax.dot_general(x, w3b, (((1,), (1,)), ((), ())),
                             preferred_element_type=jnp.float32)
    h = gate * jax.nn.sigmoid(gate) * up
    o = jax.lax.dot_general(h, w2b, (((1,), (1,)), ((), ())),
                            preferred_element_type=jnp.float32)
    o = o * ws_ref[0, 0][:, None]

    @pl.when(k == 0)
    def _():
        o_ref[...] = o

    @pl.when(k != 0)
    def _():
        o_ref[...] += o


def _grouped_mlp(xs, w1, w3, w2, ws3d, block_expert):
    def kk(i, k):
        return jax.lax.bitwise_xor(k, i % 2)

    grid_spec = pltpu.PrefetchScalarGridSpec(
        num_scalar_prefetch=1,
        grid=(NBLK, KSPLIT),
        in_specs=[
            pl.BlockSpec((BM, HIDDEN), lambda i, k, be: (i, 0)),
            pl.BlockSpec((1, IB, HIDDEN),
                         lambda i, k, be: (be[i] & 7, kk(i, k), 0)),
            pl.BlockSpec((1, IB, HIDDEN),
                         lambda i, k, be: (be[i] & 7, kk(i, k), 0)),
            pl.BlockSpec((1, HIDDEN, IB),
                         lambda i, k, be: (be[i] & 7, 0, kk(i, k))),
            pl.BlockSpec((1, 1, BM), lambda i, k, be: (i, 0, 0)),
        ],
        out_specs=pl.BlockSpec((BM, HIDDEN), lambda i, k, be: (i, 0)),
    )
    return pl.pallas_call(
        _mlp_block_kernel,
        grid_spec=grid_spec,
        out_shape=jax.ShapeDtypeStruct((PADDED, HIDDEN), jnp.float32),
        compiler_params=pltpu.CompilerParams(
            dimension_semantics=("arbitrary", "arbitrary")),
    )(block_expert, xs, w1, w3, w2, ws3d)


_NC, _NS = 2, 16
_NW = _NC * _NS
_SLOTS_PER_W = PADDED // _NW
_GCHUNK = 64
_TOK_PER_W = TOKENS // _NW
_CCHUNK = 16
_POSPAD = 2 * TOKENS + NUM_EXPERTS * 256


def _sc_mesh():
    return plsc.VectorSubcoreMesh(core_axis_name="c", subcore_axis_name="s")


def _gather_rows(x, idx, n_rows):
    per_w = n_rows // _NW
    nch = per_w // _GCHUNK

    @functools.partial(
        pl.kernel,
        out_type=jax.ShapeDtypeStruct((n_rows, HIDDEN), jnp.float32),
        mesh=_sc_mesh(),
        scratch_types=[
            pltpu.VMEM((2, _GCHUNK), jnp.int32),
            pltpu.VMEM((_GCHUNK, HIDDEN), jnp.float32),
            pltpu.VMEM((_GCHUNK, HIDDEN), jnp.float32),
            pltpu.SemaphoreType.DMA,
            pltpu.SemaphoreType.DMA,
            pltpu.SemaphoreType.DMA,
            pltpu.SemaphoreType.DMA,
        ],
    )
    def k(x_hbm, tok_hbm, xs_hbm, idx_v, rows0, rows1, g0, g1, w0, w1):
        wid = lax.axis_index("s") * _NC + lax.axis_index("c")
        base = wid * per_w
        rows = (rows0, rows1)
        gsem = (g0, g1)
        wsem = (w0, w1)
        gathers = [None] * nch
        writes = [None] * nch
        for c in range(nch):
            off = base + c * _GCHUNK
            pltpu.sync_copy(tok_hbm.at[pl.ds(off, _GCHUNK)], idx_v.at[c % 2])
            if c >= 2:
                writes[c - 2].wait()
            gathers[c] = pltpu.async_copy(
                x_hbm.at[idx_v.at[c % 2]], rows[c % 2], gsem[c % 2])
            gathers[c].wait()
            writes[c] = pltpu.async_copy(
                rows[c % 2], xs_hbm.at[pl.ds(off, _GCHUNK)], wsem[c % 2])
        for c in range(max(0, nch - 2), nch):
            writes[c].wait()

    return k(x, idx)


_POSDUMP = 2 * TOKENS
_TPT = TOKENS // _NS
_ZPT = PADDED // _NS


def _route_sort(logits_t):
    @functools.partial(
        pl.kernel,
        out_type=(
            jax.ShapeDtypeStruct((PADDED,), jnp.int32),
            jax.ShapeDtypeStruct((PADDED,), jnp.float32),
            jax.ShapeDtypeStruct((_POSPAD,), jnp.int32),
            jax.ShapeDtypeStruct((32,), jnp.int32),
        ),
        mesh=_sc_mesh(),
        compiler_params=pltpu.CompilerParams(needs_layout_passes=False),
        scratch_types=[
            pltpu.VMEM((8, _TPT), jnp.float32),
            pltpu.VMEM((_TPT,), jnp.int32),
            pltpu.VMEM((_TPT,), jnp.int32),
            pltpu.VMEM((_TPT,), jnp.float32),
            pltpu.VMEM((_TPT,), jnp.float32),
            pltpu.VMEM((_ZPT,), jnp.int32),
            pltpu.VMEM((_ZPT,), jnp.float32),
            pltpu.VMEM((2 * TOKENS,), jnp.int32),
            pltpu.VMEM((2 * TOKENS,), jnp.float32),
            pltpu.VMEM((16,), jnp.int32),
            pltpu.VMEM((2 * TOKENS // 2,), jnp.int32),
            pltpu.VMEM((2 * TOKENS // 2,), jnp.float32),
            pltpu.VMEM((16, 128), jnp.int32),
            pltpu.VMEM((16, 128), jnp.int32),
            pltpu.VMEM((32,), jnp.int32),
            pltpu.VMEM((16,), jnp.int32),
            pltpu.VMEM((16,), jnp.int32),
            pltpu.VMEM_SHARED((2 * TOKENS,), jnp.int32),
            pltpu.VMEM_SHARED((2 * TOKENS,), jnp.float32),
            pltpu.SemaphoreType.DMA,
        ],
    )
    def k(lt_hbm, tok_hbm, ws_hbm, pos_hbm, be_hbm,
          lv, e1b, e2b, p1b, p2b, zt, zw, ef_v, pf_v, acc,
          tokbuf, wsbuf, plidx, valb, bebuf, psum, off_ref,
          ef_sh, pf_sh, sem):
        cid = lax.axis_index("c")
        sid = lax.axis_index("s")
        iota = lax.iota(jnp.int32, 16)
        ones = jnp.ones((16,), jnp.int32)

        @pl.when(cid == 0)
        def _core0():
            for cc in range(_ZPT // 16):
                base = sid * _ZPT + cc * 16
                zt[pl.ds(cc * 16, 16)] = ((base + iota) * 997) & (TOKENS - 1)
                zw[pl.ds(cc * 16, 16)] = jnp.zeros((16,), jnp.float32)
            pltpu.sync_copy(zt, tok_hbm.at[pl.ds(pl.multiple_of(sid * _ZPT, 8), _ZPT)])
            pltpu.sync_copy(zw, ws_hbm.at[pl.ds(pl.multiple_of(sid * _ZPT, 8), _ZPT)])

            t0 = sid * _TPT
            pltpu.sync_copy(lt_hbm.at[:, pl.ds(t0, _TPT)], lv)
            for c8 in range(_TPT // 16):
                sl = pl.ds(c8 * 16, 16)
                l = [lv[e, sl] for e in range(NUM_EXPERTS)]
                m1 = l[0]
                for e in range(1, NUM_EXPERTS):
                    m1 = jnp.maximum(m1, l[e])
                e1 = jnp.full((16,), NUM_EXPERTS, jnp.int32)
                for e in range(NUM_EXPERTS):
                    e1 = jnp.minimum(
                        e1, jnp.where(l[e] >= m1, e, NUM_EXPERTS))
                l2 = [jnp.where(e1 == e, -1e30, l[e])
                      for e in range(NUM_EXPERTS)]
                m2 = l2[0]
                for e in range(1, NUM_EXPERTS):
                    m2 = jnp.maximum(m2, l2[e])
                e2 = jnp.full((16,), NUM_EXPERTS, jnp.int32)
                for e in range(NUM_EXPERTS):
                    e2 = jnp.minimum(
                        e2, jnp.where(l2[e] >= m2, e, NUM_EXPERTS))
                ex = jnp.exp(m2 - m1)
                p1 = 1.0 / (1.0 + ex)
                e1b[sl] = e1
                e2b[sl] = e2
                p1b[sl] = p1
                p2b[sl] = 1.0 - p1
            pltpu.sync_copy(e1b, ef_sh.at[pl.ds(pl.multiple_of(t0, 8), _TPT)])
            pltpu.sync_copy(e2b, ef_sh.at[pl.ds(pl.multiple_of(TOKENS + t0, 8), _TPT)])
            pltpu.sync_copy(p1b, pf_sh.at[pl.ds(pl.multiple_of(t0, 8), _TPT)])
            pltpu.sync_copy(p2b, pf_sh.at[pl.ds(pl.multiple_of(TOKENS + t0, 8), _TPT)])
            plsc.subcore_barrier()

            @pl.when(sid <= NUM_EXPERTS)
            def _p2a():
                pltpu.sync_copy(ef_sh, ef_v)
                pltpu.sync_copy(pf_sh, pf_v)
                acc[...] = jnp.zeros((16,), jnp.int32)

                def hbody(c, _):
                    ve = ef_v[pl.ds(pl.multiple_of(c * 16, 16), 16)]
                    plsc.addupdate_scatter(acc, [ve], ones)
                    return 0

                lax.fori_loop(0, 2 * TOKENS // 16, hbody, 0)
                cv = acc[...]
                run = jnp.int32(0)
                off_vec = jnp.zeros((16,), jnp.int32)
                for e in range(NUM_EXPERTS):
                    off_vec = off_vec + jnp.where(iota == e, run, 0)
                    run = run + (((cv[e] + (BM - 1)) >> 8) << 8)
                off_ref[...] = off_vec

                @pl.when(sid < NUM_EXPERTS)
                def _p2b():
                    sidv = jnp.zeros((16,), jnp.int32) + sid
                    my_base = plsc.load_gather(off_ref, [sidv])[0]
                    dumpb = _POSDUMP + sid * 256
                    for r in range(16):
                        for cc in range(8):
                            plidx[r, pl.ds(cc * 16, 16)] = (
                                dumpb + (r % 2) * 128 + cc * 16 + iota)
                            valb[r, pl.ds(cc * 16, 16)] = (
                                my_base + r * 128 + cc * 16 + iota)
                    for cc in range(128):
                        sl = pl.ds(cc * 16, 16)
                        tokbuf[sl] = ((my_base + cc * 16 + iota) * 997) \
                            & (TOKENS - 1)
                        wsbuf[sl] = jnp.zeros((16,), jnp.float32)

                    def sbody(c, cnt_vec):
                        sl = pl.ds(pl.multiple_of(c * 16, 16), 16)
                        ve = ef_v[sl]
                        wv = pf_v[sl]
                        tc = c * 16 + iota
                        mask = ve == sid
                        mi = jnp.where(mask, 1, 0).astype(jnp.int32)
                        ps = mi
                        for d in (1, 2, 4, 8):
                            psum[...] = ps
                            g = plsc.load_gather(
                                psum, [jnp.maximum(iota - d, 0)])
                            ps = ps + jnp.where(iota >= d, g, 0)
                        rk = ps - 1
                        slotv = cnt_vec + rk
                        tvec = tc & (TOKENS - 1)
                        plsc.store_scatter(tokbuf, [slotv], tvec, mask=mask)
                        plsc.store_scatter(wsbuf, [slotv], wv, mask=mask)
                        plsc.store_scatter(
                            plidx, [slotv >> 7, slotv & 127], tc, mask=mask)
                        psum[...] = ps
                        lane15 = jnp.full((16,), 15, jnp.int32)
                        tot = plsc.load_gather(psum, [lane15])
                        return cnt_vec + tot

                    cnt_vec = lax.fori_loop(
                        0, 2 * TOKENS // 16, sbody,
                        jnp.zeros((16,), jnp.int32))
                    cnt = cnt_vec[0]

                    def wbody(j, _):
                        src = pl.ds(pl.multiple_of(j * BM, BM), BM)
                        dst = pl.ds(
                            pl.multiple_of(my_base + j * BM, BM), BM)
                        pltpu.sync_copy(tokbuf.at[src], tok_hbm.at[dst])
                        pltpu.sync_copy(wsbuf.at[src], ws_hbm.at[dst])
                        return 0

                    lax.fori_loop(0, (cnt + BM - 1) >> 8, wbody, 0)

                    def pbody(r, _):
                        pltpu.sync_copy(valb.at[r], pos_hbm.at[plidx.at[r]])
                        return 0

                    lax.fori_loop(0, (cnt + 127) >> 7, pbody, 0)

                @pl.when(sid == NUM_EXPERTS)
                def _be():
                    for c2 in range(2):
                        blk = (iota + c2 * 16) * BM
                        accv = jnp.zeros((16,), jnp.int32)
                        for e in range(NUM_EXPERTS):
                            accv = accv + jnp.where(
                                blk >= off_vec[e], 1, 0)
                        bev = jnp.minimum(
                            jnp.maximum(accv - 1, 0), NUM_EXPERTS - 1)
                        bebuf[pl.ds(c2 * 16, 16)] = bev + jnp.where(
                            blk >= run, NUM_EXPERTS, 0)
                    pltpu.sync_copy(bebuf, be_hbm)

    return k(logits_t)


def _pair_add_kernel(a_ref, b_ref, o_ref):
    o_ref[...] = a_ref[...] + b_ref[...]


def _combine_rows(o_sorted, pos):
    pairs = _gather_rows(o_sorted, pos, 2 * TOKENS)
    badd = 512
    nb = TOKENS // badd
    return pl.pallas_call(
        _pair_add_kernel,
        grid=(nb,),
        in_specs=[
            pl.BlockSpec((badd, HIDDEN), lambda i: (i, 0)),
            pl.BlockSpec((badd, HIDDEN), lambda i: (i + nb, 0)),
        ],
        out_specs=pl.BlockSpec((badd, HIDDEN), lambda i: (i, 0)),
        out_shape=jax.ShapeDtypeStruct((TOKENS, HIDDEN), jnp.float32),
    )(pairs, pairs)


def kernel(hidden_states, router_logits, w1, w3, w2):
    x = hidden_states.reshape(-1, HIDDEN)

    tok_sorted, ws_buf, pos, block_expert = _route_sort(router_logits.T)

    xs = _gather_rows(x, tok_sorted, PADDED)

    ws3d = ws_buf.reshape(NBLK, 1, BM)
    o_sorted = _grouped_mlp(xs, w1, w3, w2, ws3d, block_expert)

    out = _combine_rows(o_sorted, pos)
    return out


# speedup vs baseline: 6.9459x; 1.0125x over previous
import functools

import jax
import jax.numpy as jnp
from jax import lax
from jax.experimental import pallas as pl
from jax.experimental.pallas import tpu as pltpu
from jax.experimental.pallas import tpu_sc as plsc

NUM_EXPERTS = 8
TOP_K = 2
HIDDEN = 768
INTER = 3072
TOKENS = 2048

BM = 256
PADDED = TOP_K * TOKENS + NUM_EXPERTS * BM
NBLK = PADDED // BM
KSPLIT = 2
IB = INTER // KSPLIT


def _mlp_block_kernel(be_ref, xs_ref, w1_ref, w3_ref, w2_ref, ws_ref, o_ref):
    k = pl.program_id(1)
    i = pl.program_id(0)
    live = be_ref[i] < NUM_EXPERTS

    @pl.when(live)
    def _body():
        _mlp_live(k, xs_ref, w1_ref, w3_ref, w2_ref, ws_ref, o_ref)


def _mlp_live(k, xs_ref, w1_ref, w3_ref, w2_ref, ws_ref, o_ref):
    x = xs_ref[...].astype(jnp.bfloat16)
    w1b = w1_ref[0].astype(jnp.bfloat16)
    w3b = w3_ref[0].astype(jnp.bfloat16)
    w2b = w2_ref[0].astype(jnp.bfloat16)
    gate = jax.lax.dot_general(x, w1b, (((1,), (1,)), ((), ())),
                               preferred_element_type=jnp.float32)
    up = jax.lax.dot_general(x, w3b, (((1,), (1,)), ((), ())),
                             preferred_element_type=jnp.float32)
    h = (gate * jax.nn.sigmoid(gate) * up).astype(jnp.bfloat16)
    o = jax.lax.dot_general(h, w2b, (((1,), (1,)), ((), ())),
                            preferred_element_type=jnp.float32)
    o = o * ws_ref[0, 0][:, None]

    @pl.when(k == 0)
    def _():
        o_ref[...] = o

    @pl.when(k != 0)
    def _():
        o_ref[...] += o


def _grouped_mlp(xs, w1, w3, w2, ws3d, block_expert):
    def kk(i, k):
        return jax.lax.bitwise_xor(k, i % 2)

    grid_spec = pltpu.PrefetchScalarGridSpec(
        num_scalar_prefetch=1,
        grid=(NBLK, KSPLIT),
        in_specs=[
            pl.BlockSpec((BM, HIDDEN), lambda i, k, be: (i, 0)),
            pl.BlockSpec((1, IB, HIDDEN),
                         lambda i, k, be: (be[i] & 7, kk(i, k), 0)),
            pl.BlockSpec((1, IB, HIDDEN),
                         lambda i, k, be: (be[i] & 7, kk(i, k), 0)),
            pl.BlockSpec((1, HIDDEN, IB),
                         lambda i, k, be: (be[i] & 7, 0, kk(i, k))),
            pl.BlockSpec((1, 1, BM), lambda i, k, be: (i, 0, 0)),
        ],
        out_specs=pl.BlockSpec((BM, HIDDEN), lambda i, k, be: (i, 0)),
    )
    return pl.pallas_call(
        _mlp_block_kernel,
        grid_spec=grid_spec,
        out_shape=jax.ShapeDtypeStruct((PADDED, HIDDEN), jnp.float32),
        compiler_params=pltpu.CompilerParams(
            dimension_semantics=("arbitrary", "arbitrary")),
    )(block_expert, xs, w1, w3, w2, ws3d)


_NC, _NS = 2, 16
_NW = _NC * _NS
_SLOTS_PER_W = PADDED // _NW
_GCHUNK = 64
_TOK_PER_W = TOKENS // _NW
_CCHUNK = 16
_POSPAD = 2 * TOKENS + NUM_EXPERTS * 256


def _sc_mesh():
    return plsc.VectorSubcoreMesh(core_axis_name="c", subcore_axis_name="s")


def _gather_rows(x, idx, n_rows):
    per_w = n_rows // _NW
    nch = per_w // _GCHUNK

    @functools.partial(
        pl.kernel,
        out_type=jax.ShapeDtypeStruct((n_rows, HIDDEN), jnp.float32),
        mesh=_sc_mesh(),
        scratch_types=[
            pltpu.VMEM((2, _GCHUNK), jnp.int32),
            pltpu.VMEM((_GCHUNK, HIDDEN), jnp.float32),
            pltpu.VMEM((_GCHUNK, HIDDEN), jnp.float32),
            pltpu.SemaphoreType.DMA,
            pltpu.SemaphoreType.DMA,
            pltpu.SemaphoreType.DMA,
            pltpu.SemaphoreType.DMA,
        ],
    )
    def k(x_hbm, tok_hbm, xs_hbm, idx_v, rows0, rows1, g0, g1, w0, w1):
        wid = lax.axis_index("s") * _NC + lax.axis_index("c")
        base = wid * per_w
        rows = (rows0, rows1)
        gsem = (g0, g1)
        wsem = (w0, w1)
        gathers = [None] * nch
        writes = [None] * nch
        for c in range(nch):
            off = base + c * _GCHUNK
            pltpu.sync_copy(tok_hbm.at[pl.ds(off, _GCHUNK)], idx_v.at[c % 2])
            if c >= 2:
                writes[c - 2].wait()
            gathers[c] = pltpu.async_copy(
                x_hbm.at[idx_v.at[c % 2]], rows[c % 2], gsem[c % 2])
            gathers[c].wait()
            writes[c] = pltpu.async_copy(
                rows[c % 2], xs_hbm.at[pl.ds(off, _GCHUNK)], wsem[c % 2])
        for c in range(max(0, nch - 2), nch):
            writes[c].wait()

    return k(x, idx)


_POSDUMP = 2 * TOKENS
_TPT = TOKENS // _NS
_ZPT = PADDED // _NS


def _route_sort(logits_t):
    @functools.partial(
        pl.kernel,
        out_type=(
            jax.ShapeDtypeStruct((PADDED,), jnp.int32),
            jax.ShapeDtypeStruct((PADDED,), jnp.float32),
            jax.ShapeDtypeStruct((_POSPAD,), jnp.int32),
            jax.ShapeDtypeStruct((32,), jnp.int32),
        ),
        mesh=_sc_mesh(),
        compiler_params=pltpu.CompilerParams(needs_layout_passes=False),
        scratch_types=[
            pltpu.VMEM((8, _TPT), jnp.float32),
            pltpu.VMEM((_TPT,), jnp.int32),
            pltpu.VMEM((_TPT,), jnp.int32),
            pltpu.VMEM((_TPT,), jnp.float32),
            pltpu.VMEM((_TPT,), jnp.float32),
            pltpu.VMEM((_ZPT,), jnp.int32),
            pltpu.VMEM((_ZPT,), jnp.float32),
            pltpu.VMEM((2 * TOKENS,), jnp.int32),
            pltpu.VMEM((2 * TOKENS,), jnp.float32),
            pltpu.VMEM((16,), jnp.int32),
            pltpu.VMEM((2 * TOKENS // 2,), jnp.int32),
            pltpu.VMEM((2 * TOKENS // 2,), jnp.float32),
            pltpu.VMEM((16, 128), jnp.int32),
            pltpu.VMEM((16, 128), jnp.int32),
            pltpu.VMEM((32,), jnp.int32),
            pltpu.VMEM((16,), jnp.int32),
            pltpu.VMEM((16,), jnp.int32),
            pltpu.VMEM_SHARED((2 * TOKENS,), jnp.int32),
            pltpu.VMEM_SHARED((2 * TOKENS,), jnp.float32),
            pltpu.SemaphoreType.DMA,
        ],
    )
    def k(lt_hbm, tok_hbm, ws_hbm, pos_hbm, be_hbm,
          lv, e1b, e2b, p1b, p2b, zt, zw, ef_v, pf_v, acc,
          tokbuf, wsbuf, plidx, valb, bebuf, psum, off_ref,
          ef_sh, pf_sh, sem):
        cid = lax.axis_index("c")
        sid = lax.axis_index("s")
        iota = lax.iota(jnp.int32, 16)
        ones = jnp.ones((16,), jnp.int32)

        @pl.when(cid == 0)
        def _core0():
            for cc in range(_ZPT // 16):
                base = sid * _ZPT + cc * 16
                zt[pl.ds(cc * 16, 16)] = ((base + iota) * 997) & (TOKENS - 1)
                zw[pl.ds(cc * 16, 16)] = jnp.zeros((16,), jnp.float32)
            pltpu.sync_copy(zt, tok_hbm.at[pl.ds(pl.multiple_of(sid * _ZPT, 8), _ZPT)])
            pltpu.sync_copy(zw, ws_hbm.at[pl.ds(pl.multiple_of(sid * _ZPT, 8), _ZPT)])

            t0 = sid * _TPT
            pltpu.sync_copy(lt_hbm.at[:, pl.ds(t0, _TPT)], lv)
            for c8 in range(_TPT // 16):
                sl = pl.ds(c8 * 16, 16)
                l = [lv[e, sl] for e in range(NUM_EXPERTS)]
                m1 = l[0]
                for e in range(1, NUM_EXPERTS):
                    m1 = jnp.maximum(m1, l[e])
                e1 = jnp.full((16,), NUM_EXPERTS, jnp.int32)
                for e in range(NUM_EXPERTS):
                    e1 = jnp.minimum(
                        e1, jnp.where(l[e] >= m1, e, NUM_EXPERTS))
                l2 = [jnp.where(e1 == e, -1e30, l[e])
                      for e in range(NUM_EXPERTS)]
                m2 = l2[0]
                for e in range(1, NUM_EXPERTS):
                    m2 = jnp.maximum(m2, l2[e])
                e2 = jnp.full((16,), NUM_EXPERTS, jnp.int32)
                for e in range(NUM_EXPERTS):
                    e2 = jnp.minimum(
                        e2, jnp.where(l2[e] >= m2, e, NUM_EXPERTS))
                ex = jnp.exp(m2 - m1)
                p1 = 1.0 / (1.0 + ex)
                e1b[sl] = e1
                e2b[sl] = e2
                p1b[sl] = p1
                p2b[sl] = 1.0 - p1
            pltpu.sync_copy(e1b, ef_sh.at[pl.ds(pl.multiple_of(t0, 8), _TPT)])
            pltpu.sync_copy(e2b, ef_sh.at[pl.ds(pl.multiple_of(TOKENS + t0, 8), _TPT)])
            pltpu.sync_copy(p1b, pf_sh.at[pl.ds(pl.multiple_of(t0, 8), _TPT)])
            pltpu.sync_copy(p2b, pf_sh.at[pl.ds(pl.multiple_of(TOKENS + t0, 8), _TPT)])
            plsc.subcore_barrier()

            @pl.when(sid <= NUM_EXPERTS)
            def _p2a():
                pltpu.sync_copy(ef_sh, ef_v)
                pltpu.sync_copy(pf_sh, pf_v)
                acc[...] = jnp.zeros((16,), jnp.int32)

                def hbody(c, _):
                    ve = ef_v[pl.ds(pl.multiple_of(c * 16, 16), 16)]
                    plsc.addupdate_scatter(acc, [ve], ones)
                    return 0

                lax.fori_loop(0, 2 * TOKENS // 16, hbody, 0)
                cv = acc[...]
                run = jnp.int32(0)
                off_vec = jnp.zeros((16,), jnp.int32)
                for e in range(NUM_EXPERTS):
                    off_vec = off_vec + jnp.where(iota == e, run, 0)
                    run = run + (((cv[e] + (BM - 1)) >> 8) << 8)
                off_ref[...] = off_vec

                @pl.when(sid < NUM_EXPERTS)
                def _p2b():
                    sidv = jnp.zeros((16,), jnp.int32) + sid
                    my_base = plsc.load_gather(off_ref, [sidv])[0]
                    dumpb = _POSDUMP + sid * 256
                    for r in range(16):
                        for cc in range(8):
                            plidx[r, pl.ds(cc * 16, 16)] = (
                                dumpb + (r % 2) * 128 + cc * 16 + iota)
                            valb[r, pl.ds(cc * 16, 16)] = (
                                my_base + r * 128 + cc * 16 + iota)
                    for cc in range(128):
                        sl = pl.ds(cc * 16, 16)
                        tokbuf[sl] = ((my_base + cc * 16 + iota) * 997) \
                            & (TOKENS - 1)
                        wsbuf[sl] = jnp.zeros((16,), jnp.float32)

                    def sbody(c, cnt_vec):
                        sl = pl.ds(pl.multiple_of(c * 16, 16), 16)
                        ve = ef_v[sl]
                        wv = pf_v[sl]
                        tc = c * 16 + iota
                        mask = ve == sid
                        mi = jnp.where(mask, 1, 0).astype(jnp.int32)
                        ps = mi
                        for d in (1, 2, 4, 8):
                            psum[...] = ps
                            g = plsc.load_gather(
                                psum, [jnp.maximum(iota - d, 0)])
                            ps = ps + jnp.where(iota >= d, g, 0)
                        rk = ps - 1
                        slotv = cnt_vec + rk
                        tvec = tc & (TOKENS - 1)
                        plsc.store_scatter(tokbuf, [slotv], tvec, mask=mask)
                        plsc.store_scatter(wsbuf, [slotv], wv, mask=mask)
                        plsc.store_scatter(
                            plidx, [slotv >> 7, slotv & 127], tc, mask=mask)
                        psum[...] = ps
                        lane15 = jnp.full((16,), 15, jnp.int32)
                        tot = plsc.load_gather(psum, [lane15])
                        return cnt_vec + tot

                    cnt_vec = lax.fori_loop(
                        0, 2 * TOKENS // 16, sbody,
                        jnp.zeros((16,), jnp.int32))
                    cnt = cnt_vec[0]

                    def wbody(j, _):
                        src = pl.ds(pl.multiple_of(j * BM, BM), BM)
                        dst = pl.ds(
                            pl.multiple_of(my_base + j * BM, BM), BM)
                        pltpu.sync_copy(tokbuf.at[src], tok_hbm.at[dst])
                        pltpu.sync_copy(wsbuf.at[src], ws_hbm.at[dst])
                        return 0

                    lax.fori_loop(0, (cnt + BM - 1) >> 8, wbody, 0)

                    def pbody(r, _):
                        pltpu.sync_copy(valb.at[r], pos_hbm.at[plidx.at[r]])
                        return 0

                    lax.fori_loop(0, (cnt + 127) >> 7, pbody, 0)

                @pl.when(sid == NUM_EXPERTS)
                def _be():
                    for c2 in range(2):
                        blk = (iota + c2 * 16) * BM
                        accv = jnp.zeros((16,), jnp.int32)
                        for e in range(NUM_EXPERTS):
                            accv = accv + jnp.where(
                                blk >= off_vec[e], 1, 0)
                        bev = jnp.minimum(
                            jnp.maximum(accv - 1, 0), NUM_EXPERTS - 1)
                        bebuf[pl.ds(c2 * 16, 16)] = bev + jnp.where(
                            blk >= run, NUM_EXPERTS, 0)
                    pltpu.sync_copy(bebuf, be_hbm)

    return k(logits_t)


def _pair_add_kernel(a_ref, b_ref, o_ref):
    o_ref[...] = a_ref[...] + b_ref[...]


def _combine_rows(o_sorted, pos):
    pairs = _gather_rows(o_sorted, pos, 2 * TOKENS)
    badd = 512
    nb = TOKENS // badd
    return pl.pallas_call(
        _pair_add_kernel,
        grid=(nb,),
        in_specs=[
            pl.BlockSpec((badd, HIDDEN), lambda i: (i, 0)),
            pl.BlockSpec((badd, HIDDEN), lambda i: (i + nb, 0)),
        ],
        out_specs=pl.BlockSpec((badd, HIDDEN), lambda i: (i, 0)),
        out_shape=jax.ShapeDtypeStruct((TOKENS, HIDDEN), jnp.float32),
    )(pairs, pairs)


def kernel(hidden_states, router_logits, w1, w3, w2):
    x = hidden_states.reshape(-1, HIDDEN)

    tok_sorted, ws_buf, pos, block_expert = _route_sort(router_logits.T)

    xs = _gather_rows(x, tok_sorted, PADDED)

    ws3d = ws_buf.reshape(NBLK, 1, BM)
    o_sorted = _grouped_mlp(xs, w1, w3, w2, ws3d, block_expert)

    out = _combine_rows(o_sorted, pos)
    return out
